# Initial kernel scaffold; baseline (speedup 1.0000x reference)
import jax
import jax.numpy as jnp
from jax.experimental import pallas as pl


def kernel(x_feat, static_adj, static_weight, dynamic_weight, w_fc, w_tr, b_tr, w_g, b_g, bn_gamma, bn_beta, w_co, b_co, w_last, b_last):
    raise NotImplementedError("write your pallas kernel here")


# speedup vs baseline: 2.1427x; 2.1427x over previous
import jax
import jax.numpy as jnp
from jax import lax
from jax.experimental import pallas as pl
from jax.experimental.pallas import tpu as pltpu

NEG_SLOPE = 0.2
BN_EPS = 1e-5
BB = 8
HW = 49
ROWS = 56


def _leaky(x):
    return jnp.where(x >= 0, x, NEG_SLOPE * x)


def _kern_a(x_ref, wcat_ref, btr_ref, adjn_ref, ws_ref,
            out1_ref, vt_ref, h_ref, xglb_ref, t_scr, *, n_nodes, n_pad, d):
    s = jnp.dot(x_ref[...], wcat_ref[...],
                preferred_element_type=jnp.float32)
    for i in range(BB):
        blk = s[i * ROWS:(i + 1) * ROWS, :]
        sc = blk[:, :n_pad]
        row = lax.broadcasted_iota(jnp.int32, (ROWS, n_pad), 0)
        valid = row < HW
        out1_ref[i:i + 1, :] = jnp.max(
            jnp.where(valid, sc, -jnp.inf), axis=0, keepdims=True)
        mask = jnp.where(valid, jax.nn.sigmoid(sc), 0.0)
        xt = blk[:, n_pad:] + btr_ref[...]
        vt_i = lax.dot_general(mask.astype(jnp.bfloat16),
                               xt.astype(jnp.bfloat16),
                               (((0,), (0,)), ((), ())),
                               preferred_element_type=jnp.float32)
        vt_ref[i * n_nodes:(i + 1) * n_nodes, :] = vt_i[:n_nodes, :]
    v_all = vt_ref[...]
    for i in range(BB):
        t_i = jnp.dot(adjn_ref[...],
                      v_all[i * n_nodes:(i + 1) * n_nodes, :]
                      .astype(jnp.bfloat16),
                      preferred_element_type=jnp.float32)
        t_scr[i * n_nodes:(i + 1) * n_nodes, :] = \
            _leaky(t_i).astype(jnp.bfloat16)
    h_all = v_all + jnp.dot(t_scr[...], ws_ref[...],
                            preferred_element_type=jnp.float32)
    h_ref[...] = h_all
    xglb_ref[...] = jnp.mean(h_all.reshape(BB, n_nodes, d), axis=1)


def _kern_b(h_ref, vt_ref, g_ref, wcog_ref, wcox_ref, bco_ref, sadj_ref,
            wdyn_ref, wlast_ref, blast_ref, out2_ref, t_scr, *, n_nodes, d):
    tg = lax.dot_general(wcog_ref[...], g_ref[...].astype(jnp.bfloat16),
                         (((1,), (1,)), ((), ())),
                         preferred_element_type=jnp.float32)
    for i in range(BB):
        h_i = h_ref[i * n_nodes:(i + 1) * n_nodes, :]
        h_bf = h_i.astype(jnp.bfloat16)
        tx = lax.dot_general(wcox_ref[...], h_bf,
                             (((1,), (1,)), ((), ())),
                             preferred_element_type=jnp.float32)
        a = jax.nn.sigmoid(tx + tg[:, i:i + 1] + bco_ref[...])
        a = (a + sadj_ref[...]) * 0.5
        dv = lax.rsqrt(jnp.sum(a, axis=1, keepdims=True))
        m = (dv * h_i).astype(jnp.bfloat16)
        t_i = lax.dot_general(a.astype(jnp.bfloat16), m,
                              (((0,), (0,)), ((), ())),
                              preferred_element_type=jnp.float32)
        t_scr[i * n_nodes:(i + 1) * n_nodes, :] = \
            _leaky(dv * t_i).astype(jnp.bfloat16)
    z = _leaky(jnp.dot(t_scr[...], wdyn_ref[...],
                       preferred_element_type=jnp.float32))
    y = vt_ref[...] + z
    yw = y.reshape(BB, n_nodes, d) * wlast_ref[...][None]
    out2_ref[...] = jnp.sum(yw, axis=2) + blast_ref[...]


def kernel(x_feat, static_adj, static_weight, dynamic_weight, w_fc, w_tr,
           b_tr, w_g, b_g, bn_gamma, bn_beta, w_co, b_co, w_last, b_last):
    B, Cf, H, W = x_feat.shape
    n_nodes = w_fc.shape[0]
    d = w_tr.shape[0]
    n_pad = 128
    nblk = B // BB

    xb = x_feat.reshape(B, Cf, H * W).astype(jnp.bfloat16)
    xt = jnp.pad(xb.transpose(0, 2, 1), ((0, 0), (0, ROWS - HW), (0, 0)))
    xt = xt.reshape(B * ROWS, Cf)
    wcat = jnp.concatenate(
        [w_fc, jnp.zeros((n_pad - n_nodes, Cf), jnp.float32), w_tr], axis=0)
    wcat_t = wcat.T.astype(jnp.bfloat16)
    A = static_adj
    dv = jnp.sum(A, axis=1) ** -0.5
    adjn = (dv[:, None] * A.T * dv[None, :]).astype(jnp.bfloat16)
    ws_bf = static_weight.astype(jnp.bfloat16)
    btr_row = b_tr.reshape(1, d)

    out1f, vt, h, xglb = pl.pallas_call(
        lambda *refs: _kern_a(*refs, n_nodes=n_nodes, n_pad=n_pad, d=d),
        grid=(nblk,),
        in_specs=[
            pl.BlockSpec((BB * ROWS, Cf), lambda i: (i, 0)),
            pl.BlockSpec((Cf, n_pad + d), lambda i: (0, 0)),
            pl.BlockSpec((1, d), lambda i: (0, 0)),
            pl.BlockSpec((n_nodes, n_nodes), lambda i: (0, 0)),
            pl.BlockSpec((d, d), lambda i: (0, 0)),
        ],
        out_specs=[
            pl.BlockSpec((BB, n_pad), lambda i: (i, 0)),
            pl.BlockSpec((BB * n_nodes, d), lambda i: (i, 0)),
            pl.BlockSpec((BB * n_nodes, d), lambda i: (i, 0)),
            pl.BlockSpec((BB, d), lambda i: (i, 0)),
        ],
        out_shape=[
            jax.ShapeDtypeStruct((B, n_pad), jnp.float32),
            jax.ShapeDtypeStruct((B * n_nodes, d), jnp.float32),
            jax.ShapeDtypeStruct((B * n_nodes, d), jnp.float32),
            jax.ShapeDtypeStruct((B, d), jnp.float32),
        ],
        scratch_shapes=[pltpu.VMEM((BB * n_nodes, d), jnp.bfloat16)],
        compiler_params=pltpu.CompilerParams(
            dimension_semantics=("parallel",)),
    )(xt, wcat_t, btr_row, adjn, ws_bf)
    out1 = out1f[:, :n_nodes]

    y = xglb @ w_g.T + b_g
    mu = jnp.mean(y, axis=0, keepdims=True)
    var = jnp.mean((y - mu) ** 2, axis=0, keepdims=True)
    g = _leaky((y - mu) * lax.rsqrt(var + BN_EPS) * bn_gamma + bn_beta)

    out2 = pl.pallas_call(
        lambda *refs: _kern_b(*refs, n_nodes=n_nodes, d=d),
        grid=(nblk,),
        in_specs=[
            pl.BlockSpec((BB * n_nodes, d), lambda i: (i, 0)),
            pl.BlockSpec((BB * n_nodes, d), lambda i: (i, 0)),
            pl.BlockSpec((BB, d), lambda i: (i, 0)),
            pl.BlockSpec((n_nodes, d), lambda i: (0, 0)),
            pl.BlockSpec((n_nodes, d), lambda i: (0, 0)),
            pl.BlockSpec((n_nodes, 1), lambda i: (0, 0)),
            pl.BlockSpec((n_nodes, n_nodes), lambda i: (0, 0)),
            pl.BlockSpec((d, d), lambda i: (0, 0)),
            pl.BlockSpec((n_nodes, d), lambda i: (0, 0)),
            pl.BlockSpec((1, n_nodes), lambda i: (0, 0)),
        ],
        out_specs=pl.BlockSpec((BB, n_nodes), lambda i: (i, 0)),
        out_shape=jax.ShapeDtypeStruct((B, n_nodes), jnp.float32),
        scratch_shapes=[pltpu.VMEM((BB * n_nodes, d), jnp.bfloat16)],
        compiler_params=pltpu.CompilerParams(
            dimension_semantics=("parallel",)),
    )(h, vt, g, w_co[:, :d].astype(jnp.bfloat16),
      w_co[:, d:].astype(jnp.bfloat16), b_co, static_adj,
      dynamic_weight.astype(jnp.bfloat16), w_last, b_last.reshape(1, n_nodes))
    return out1, out2
